# Initial kernel scaffold; baseline (speedup 1.0000x reference)
import jax
import jax.numpy as jnp
from jax.experimental import pallas as pl


def kernel(x, W):
    raise NotImplementedError("write your pallas kernel here")


# speedup vs baseline: 1.3396x; 1.3396x over previous
import functools

import jax
import jax.numpy as jnp
from jax.experimental import pallas as pl

_B, _S, _T, _D, _E, _TOP_K = 1, 4, 2048, 8192, 64, 2
_TILE = 256


def _gate_kernel(x_ref, w_ref, out_ref):
    logits = jax.lax.dot_general(
        x_ref[...], w_ref[...],
        dimension_numbers=(((1,), (1,)), ((), ())),
        preferred_element_type=jnp.float32,
    )

    lane = jax.lax.broadcasted_iota(jnp.int32, (_TILE, _E), 1)

    m1 = jnp.max(logits, axis=-1, keepdims=True)
    a1 = jnp.argmax(logits, axis=-1, keepdims=True)
    neg_inf = jnp.float32(-jnp.inf)
    masked = jnp.where(lane == a1, neg_inf, logits)
    m2 = jnp.max(masked, axis=-1, keepdims=True)
    a2 = jnp.argmax(masked, axis=-1, keepdims=True)

    e2 = jnp.exp(m2 - m1)
    denom = 1.0 + e2
    w1 = 1.0 / denom
    w2 = e2 / denom

    zero = jnp.float32(0.0)
    out = jnp.where(lane == a1, w1, zero) + jnp.where(lane == a2, w2, zero)
    out_ref[...] = out


@jax.jit
def kernel(x, W):
    n_tok = _B * _S * _T
    x2 = x.reshape(n_tok, _D)
    grid = (n_tok // _TILE,)
    out = pl.pallas_call(
        _gate_kernel,
        grid=grid,
        in_specs=[
            pl.BlockSpec((_TILE, _D), lambda i: (i, 0)),
            pl.BlockSpec((_E, _D), lambda i: (0, 0)),
        ],
        out_specs=pl.BlockSpec((_TILE, _E), lambda i: (i, 0)),
        out_shape=jax.ShapeDtypeStruct((n_tok, _E), jnp.float32),
    )(x2, W)
    return out.reshape(_B, _S, _T, _E)


# speedup vs baseline: 1.3429x; 1.0025x over previous
import functools

import jax
import jax.numpy as jnp
from jax.experimental import pallas as pl

_B, _S, _T, _D, _E, _TOP_K = 1, 4, 2048, 8192, 64, 2
_TILE = 512


def _gate_kernel(x_ref, w_ref, out_ref):
    logits = jax.lax.dot_general(
        x_ref[...], w_ref[...],
        dimension_numbers=(((1,), (1,)), ((), ())),
        preferred_element_type=jnp.float32,
    )

    lane = jax.lax.broadcasted_iota(jnp.int32, (_TILE, _E), 1)

    m1 = jnp.max(logits, axis=-1, keepdims=True)
    a1 = jnp.argmax(logits, axis=-1, keepdims=True)
    neg_inf = jnp.float32(-jnp.inf)
    masked = jnp.where(lane == a1, neg_inf, logits)
    m2 = jnp.max(masked, axis=-1, keepdims=True)
    a2 = jnp.argmax(masked, axis=-1, keepdims=True)

    e2 = jnp.exp(m2 - m1)
    denom = 1.0 + e2
    w1 = 1.0 / denom
    w2 = e2 / denom

    zero = jnp.float32(0.0)
    out = jnp.where(lane == a1, w1, zero) + jnp.where(lane == a2, w2, zero)
    out_ref[...] = out


@jax.jit
def kernel(x, W):
    n_tok = _B * _S * _T
    x2 = x.reshape(n_tok, _D)
    grid = (n_tok // _TILE,)
    out = pl.pallas_call(
        _gate_kernel,
        grid=grid,
        in_specs=[
            pl.BlockSpec((_TILE, _D), lambda i: (i, 0)),
            pl.BlockSpec((_E, _D), lambda i: (0, 0)),
        ],
        out_specs=pl.BlockSpec((_TILE, _E), lambda i: (i, 0)),
        out_shape=jax.ShapeDtypeStruct((n_tok, _E), jnp.float32),
    )(x2, W)
    return out.reshape(_B, _S, _T, _E)
